# baseline (speedup 1.0000x reference)
import functools

import jax
import jax.numpy as jnp
from jax import lax
from jax.experimental import pallas as pl
from jax.experimental.pallas import tpu as pltpu
from jax.experimental.pallas import tpu_sc as plsc

_LANE = 128
_D = 64
_RING = 3
_UNROLL = 6


def _encoder_gather(widx_raw, widx_half, p_idx, wtab2, ptab):
    nw, g_per_w = widx_raw.shape[0], widx_raw.shape[1]
    n_groups = nw * g_per_w
    info = plsc.get_sparse_core_info()
    mesh = plsc.VectorSubcoreMesh(core_axis_name="c", subcore_axis_name="s")

    @functools.partial(
        pl.kernel,
        mesh=mesh,
        out_type=jax.ShapeDtypeStruct((n_groups * _LANE, 2 * _D), jnp.float32),
        scratch_types=[
            pltpu.VMEM((g_per_w, _LANE), jnp.int32),
            pltpu.VMEM((g_per_w, _LANE), jnp.int32),
            pltpu.VMEM((g_per_w, _LANE), jnp.int32),
            pltpu.VMEM((_RING, _LANE, 2 * _D), jnp.float32),
            pltpu.VMEM((2, _LANE, 2 * _D), jnp.float32),
            pltpu.VMEM_SHARED((1600, 2 * _D), jnp.float32),
            pltpu.SemaphoreType.DMA((_RING,)),
            pltpu.SemaphoreType.DMA((2,)),
            pltpu.SemaphoreType.DMA((_RING,)),
        ],
    )
    def k(widx_hbm, wh_hbm, pidx_hbm, wtab_hbm, ptab_hbm, out_hbm,
          widx_v, wh_v, pidx_v, wbuf_v, gp_v, ptab_s, sem_w, sem_p, sem_o):
        wid = lax.axis_index("s") * info.num_cores + lax.axis_index("c")
        g0 = wid * g_per_w

        @pl.when(lax.axis_index("s") == 0)
        def _stage():
            pltpu.sync_copy(ptab_hbm, ptab_s)

        pltpu.sync_copy(widx_hbm.at[wid], widx_v)
        pltpu.sync_copy(wh_hbm.at[wid], wh_v)
        pltpu.sync_copy(pidx_hbm.at[wid], pidx_v)
        plsc.subcore_barrier()

        def fire_p(j, b2):
            pltpu.async_copy(ptab_s.at[pidx_v.at[j]], gp_v.at[b2],
                             sem_p.at[b2])

        def fire_w(j, b):
            pltpu.async_copy(wtab_hbm.at[wh_v.at[j]], wbuf_v.at[b],
                             sem_w.at[b])

        def out_slice(j):
            return out_hbm.at[pl.ds((g0 + j) * _LANE, _LANE)]

        def wait(sem, b, j):
            pltpu.make_async_copy(wbuf_v.at[b], out_slice(j), sem.at[b]).wait()

        fire_w(0, 0)
        fire_w(1, 1)
        fire_p(0, 0)

        def step(j, b, b2):
            bp = (b + 2) % _RING

            @pl.when(j + 2 < g_per_w)
            def _fire_word():
                @pl.when(j >= _RING - 2)
                def _drain_old_write():
                    wait(sem_o, bp, j)
                fire_w(j + 2, bp)

            @pl.when(j + 1 < g_per_w)
            def _fire_pitch():
                fire_p(j + 1, 1 - b2)

            wait(sem_w, b, j)
            wait(sem_p, b2, j)

            def patch(r16, c2):
                base = r16 * 16
                parv = (widx_v[j, pl.ds(base, 16)] & 1) * _D
                for rr in range(16):
                    r = base + rr
                    col0 = parv[rr]
                    for kk in range(_D // 16):
                        wbuf_v[b, r, pl.ds(16 * kk, 16)] = (
                            wbuf_v[b, r, pl.ds(col0 + 16 * kk, 16)])
                    for kk in range(_D // 16):
                        col = _D + 16 * kk
                        wbuf_v[b, r, pl.ds(col, 16)] = (
                            gp_v[b2, r, pl.ds(col, 16)])
                return c2

            lax.fori_loop(0, _LANE // 16, patch, 0)
            pltpu.async_copy(wbuf_v.at[b], out_slice(j), sem_o.at[b])

        n_main = (g_per_w // _UNROLL) * _UNROLL

        def body(i, carry):
            for bb in range(_UNROLL):
                step(_UNROLL * i + bb, bb % _RING, bb % 2)
            return carry

        lax.fori_loop(0, n_main // _UNROLL, body, 0)
        for j in range(n_main, g_per_w):
            step(j, j % _RING, j % 2)
        for j in range(g_per_w - _RING, g_per_w):
            wait(sem_o, j % _RING, j)

    return k(widx_raw, widx_half, p_idx, wtab2, ptab)


def kernel(src_seq, mask, d_seq, p_seq, src_word_emb, pitch_emb):
    B, L = src_seq.shape
    src_idx = src_seq.reshape(32, -1, _LANE).astype(jnp.int32)
    src_half = src_idx >> 1
    p_idx = p_seq.reshape(32, -1, _LANE).astype(jnp.int32)
    wtab2 = src_word_emb.reshape(-1, 2 * _D)
    ptab = jnp.pad(pitch_emb, ((0, 0), (_D, 0)))
    out = _encoder_gather(src_idx, src_half, p_idx, wtab2, ptab)
    return out.reshape(B, L, 2 * _D)


# speedup vs baseline: 1.4414x; 1.4414x over previous
import functools

import jax
import jax.numpy as jnp
from jax import lax
from jax.experimental import pallas as pl
from jax.experimental.pallas import tpu as pltpu
from jax.experimental.pallas import tpu_sc as plsc

_LANE = 128
_D = 64
_RING = 4


def _encoder_gather(src_idx, p_idx, wtab, ptab):
    nw, g_per_w = src_idx.shape[0], src_idx.shape[1]
    n_groups = nw * g_per_w
    info = plsc.get_sparse_core_info()
    mesh = plsc.VectorSubcoreMesh(core_axis_name="c", subcore_axis_name="s")

    @functools.partial(
        pl.kernel,
        mesh=mesh,
        out_type=jax.ShapeDtypeStruct((n_groups, _LANE, 2 * _D), jnp.float32),
        scratch_types=[
            pltpu.VMEM((g_per_w, _LANE), jnp.int32),
            pltpu.VMEM((g_per_w, _LANE), jnp.int32),
            pltpu.VMEM((2, 2, _LANE, 2 * _D), jnp.float32),
            pltpu.VMEM((2, _LANE, 2 * _D), jnp.float32),
            pltpu.VMEM_SHARED((1600, 2 * _D), jnp.float32),
            pltpu.SemaphoreType.DMA((2, 2)),
            pltpu.SemaphoreType.DMA((2,)),
            pltpu.SemaphoreType.DMA((2,)),
        ],
    )
    def k(widx_hbm, pidx_hbm, wtab_hbm, ptab_hbm, out_hbm,
          widx_v, pidx_v, wbuf_v, gp_v, ptab_s, sem_w, sem_p, sem_o):
        wid = lax.axis_index("s") * info.num_cores + lax.axis_index("c")
        g0 = wid * g_per_w

        @pl.when(lax.axis_index("s") == 0)
        def _stage():
            pltpu.sync_copy(ptab_hbm, ptab_s)

        pltpu.sync_copy(widx_hbm.at[wid], widx_v)
        pltpu.sync_copy(pidx_hbm.at[wid], pidx_v)
        plsc.subcore_barrier()

        def fire_p(j, s):
            pltpu.async_copy(ptab_s.at[pidx_v.at[j]], gp_v.at[s], sem_p.at[s])

        def fire_w(j, pb, s):
            pltpu.async_copy(wtab_hbm.at[widx_v.at[j]], wbuf_v.at[pb, s],
                             sem_w.at[pb, s])

        def wait_w(j, pb, s):
            pltpu.make_async_copy(wbuf_v.at[pb, s], out_hbm.at[0],
                                  sem_w.at[pb, s]).wait()

        def wait_p(j, s):
            pltpu.make_async_copy(gp_v.at[s], out_hbm.at[0],
                                  sem_p.at[s]).wait()

        def wait_o(pb):
            pltpu.make_async_copy(wbuf_v.at[pb], out_hbm.at[pl.ds(0, 2)],
                                  sem_o.at[pb]).wait()

        fire_w(0, 0, 0)
        fire_w(1, 0, 1)
        fire_p(0, 0)

        def step(j, pb, s):
            @pl.when(j + 2 < g_per_w)
            def _fire_word():
                if s == 0:
                    @pl.when(j >= 2)
                    def _drain_old_write():
                        wait_o(1 - pb)
                fire_w(j + 2, 1 - pb, s)

            @pl.when(j + 1 < g_per_w)
            def _fire_pitch():
                fire_p(j + 1, 1 - s)

            wait_w(j, pb, s)
            wait_p(j, s)

            def patch(r4, c2):
                for rr in range(4):
                    r = r4 * 4 + rr
                    for kk in range(_D // 16):
                        col = _D + 16 * kk
                        wbuf_v[pb, s, r, pl.ds(col, 16)] = (
                            gp_v[s, r, pl.ds(col, 16)])
                return c2

            lax.fori_loop(0, _LANE // 4, patch, 0)
            if s == 1:
                pltpu.async_copy(wbuf_v.at[pb],
                                 out_hbm.at[pl.ds(g0 + j - 1, 2)],
                                 sem_o.at[pb])

        n_main = (g_per_w // 4) * 4

        def body(i, carry):
            for bb in range(4):
                step(4 * i + bb, (bb // 2) % 2, bb % 2)
            return carry

        lax.fori_loop(0, n_main // 4, body, 0)
        for j in range(n_main, g_per_w):
            step(j, (j // 2) % 2, j % 2)
        wait_o(((g_per_w - 2) // 2) % 2)
        wait_o(((g_per_w - 4) // 2) % 2)

    return k(src_idx, p_idx, wtab, ptab)


def kernel(src_seq, mask, d_seq, p_seq, src_word_emb, pitch_emb):
    B, L = src_seq.shape
    src_idx = src_seq.reshape(32, -1, _LANE).astype(jnp.int32)
    p_idx = p_seq.reshape(32, -1, _LANE).astype(jnp.int32)
    wtab = jnp.pad(src_word_emb, ((0, 0), (0, _D)))
    ptab = jnp.pad(pitch_emb, ((0, 0), (_D, 0)))
    out = _encoder_gather(src_idx, p_idx, wtab, ptab)
    return out.reshape(B, L, 2 * _D)


# speedup vs baseline: 1.4458x; 1.0031x over previous
import functools

import jax
import jax.numpy as jnp
from jax import lax
from jax.experimental import pallas as pl
from jax.experimental.pallas import tpu as pltpu
from jax.experimental.pallas import tpu_sc as plsc

_LANE = 128
_D = 64
_RING = 4


def _encoder_gather(src_idx, p_idx, wtab, ptab):
    nw, g_per_w = src_idx.shape[0], src_idx.shape[1]
    n_groups = nw * g_per_w
    info = plsc.get_sparse_core_info()
    mesh = plsc.VectorSubcoreMesh(core_axis_name="c", subcore_axis_name="s")

    @functools.partial(
        pl.kernel,
        mesh=mesh,
        out_type=jax.ShapeDtypeStruct((n_groups * _LANE, 2 * _D), jnp.float32),
        scratch_types=[
            pltpu.VMEM((g_per_w, _LANE), jnp.int32),
            pltpu.VMEM((g_per_w, _LANE), jnp.int32),
            pltpu.VMEM((_RING, _LANE, 2 * _D), jnp.float32),
            pltpu.VMEM((2, _LANE, 2 * _D), jnp.float32),
            pltpu.VMEM_SHARED((1600, 2 * _D), jnp.float32),
            pltpu.SemaphoreType.DMA((_RING,)),
            pltpu.SemaphoreType.DMA((2,)),
            pltpu.SemaphoreType.DMA((_RING,)),
        ],
    )
    def k(widx_hbm, pidx_hbm, wtab_hbm, ptab_hbm, out_hbm,
          widx_v, pidx_v, wbuf_v, gp_v, ptab_s, sem_w, sem_p, sem_o):
        wid = lax.axis_index("s") * info.num_cores + lax.axis_index("c")
        g0 = wid * g_per_w

        @pl.when(lax.axis_index("s") == 0)
        def _stage():
            pltpu.sync_copy(ptab_hbm, ptab_s)

        pltpu.sync_copy(widx_hbm.at[wid], widx_v)
        pltpu.sync_copy(pidx_hbm.at[wid], pidx_v)
        plsc.subcore_barrier()

        def fire_p(j, b2):
            pltpu.async_copy(ptab_s.at[pidx_v.at[j]], gp_v.at[b2],
                             sem_p.at[b2])

        def fire_w(j, b):
            pltpu.async_copy(wtab_hbm.at[widx_v.at[j]], wbuf_v.at[b],
                             sem_w.at[b])

        def out_slice(j):
            return out_hbm.at[pl.ds((g0 + j) * _LANE, _LANE)]

        def wait(sem, b, j):
            pltpu.make_async_copy(wbuf_v.at[b], out_slice(j), sem.at[b]).wait()

        fire_w(0, 0)
        fire_w(1, 1)
        fire_p(0, 0)

        def step(j, b, b2):
            bp = (b + 2) % _RING

            @pl.when(j + 2 < g_per_w)
            def _fire_word():
                @pl.when(j >= _RING - 2)
                def _drain_old_write():
                    wait(sem_o, bp, j)
                fire_w(j + 2, bp)

            @pl.when(j + 1 < g_per_w)
            def _fire_pitch():
                fire_p(j + 1, 1 - b2)

            wait(sem_w, b, j)
            wait(sem_p, b2, j)

            def patch(r4, c2):
                for rr in range(4):
                    r = r4 * 4 + rr
                    for kk in range(_D // 16):
                        col = _D + 16 * kk
                        wbuf_v[b, r, pl.ds(col, 16)] = (
                            gp_v[b2, r, pl.ds(col, 16)])
                return c2

            lax.fori_loop(0, _LANE // 4, patch, 0)
            pltpu.async_copy(wbuf_v.at[b], out_slice(j), sem_o.at[b])

        n_main = (g_per_w // _RING) * _RING

        def body(i, carry):
            for bb in range(_RING):
                step(_RING * i + bb, bb, bb % 2)
            return carry

        lax.fori_loop(0, n_main // _RING, body, 0)
        for j in range(n_main, g_per_w):
            step(j, j % _RING, j % 2)
        for j in range(g_per_w - _RING, g_per_w):
            wait(sem_o, j % _RING, j)

    return k(src_idx, p_idx, wtab, ptab)


def kernel(src_seq, mask, d_seq, p_seq, src_word_emb, pitch_emb):
    B, L = src_seq.shape
    src_idx = src_seq.reshape(32, -1, _LANE).astype(jnp.int32)
    p_idx = p_seq.reshape(32, -1, _LANE).astype(jnp.int32)
    wtab = jnp.pad(src_word_emb, ((0, 0), (0, _D)))
    ptab = jnp.pad(pitch_emb, ((0, 0), (_D, 0)))
    out = _encoder_gather(src_idx, p_idx, wtab, ptab)
    return out.reshape(B, L, 2 * _D)


# speedup vs baseline: 1.4538x; 1.0055x over previous
import functools

import jax
import jax.numpy as jnp
from jax import lax
from jax.experimental import pallas as pl
from jax.experimental.pallas import tpu as pltpu
from jax.experimental.pallas import tpu_sc as plsc

_LANE = 128
_D = 64
_RING = 4


def _encoder_gather(src_idx, p_idx, tab, n_pitch):
    nw, g_per_w = src_idx.shape[0], src_idx.shape[1]
    n_groups = nw * g_per_w
    n_word = tab.shape[0] - n_pitch
    info = plsc.get_sparse_core_info()
    mesh = plsc.VectorSubcoreMesh(core_axis_name="c", subcore_axis_name="s")

    @functools.partial(
        pl.kernel,
        mesh=mesh,
        out_type=jax.ShapeDtypeStruct((n_groups * _LANE, 2 * _D), jnp.float32),
        scratch_types=[
            pltpu.VMEM((g_per_w, _LANE), jnp.int32),
            pltpu.VMEM((g_per_w, _LANE), jnp.int32),
            pltpu.VMEM((_RING, _LANE, 2 * _D), jnp.float32),
            pltpu.VMEM((2, _LANE, 2 * _D), jnp.float32),
            pltpu.VMEM_SHARED((n_pitch, 2 * _D), jnp.float32),
            pltpu.SemaphoreType.DMA((_RING,)),
            pltpu.SemaphoreType.DMA((2,)),
            pltpu.SemaphoreType.DMA((_RING,)),
        ],
    )
    def k(widx_hbm, pidx_hbm, tab_hbm, out_hbm,
          widx_v, pidx_v, wbuf_v, gp_v, ptab_s, sem_w, sem_p, sem_o):
        wid = lax.axis_index("s") * info.num_cores + lax.axis_index("c")
        g0 = wid * g_per_w

        @pl.when(lax.axis_index("s") == 0)
        def _stage():
            pltpu.sync_copy(tab_hbm.at[pl.ds(n_word, n_pitch)], ptab_s)

        pltpu.sync_copy(widx_hbm.at[wid], widx_v)
        pltpu.sync_copy(pidx_hbm.at[wid], pidx_v)
        plsc.subcore_barrier()

        def fire_p(j, b2):
            pltpu.async_copy(ptab_s.at[pidx_v.at[j]], gp_v.at[b2],
                             sem_p.at[b2])

        def fire_w(j, b):
            pltpu.async_copy(tab_hbm.at[widx_v.at[j]], wbuf_v.at[b],
                             sem_w.at[b])

        def out_slice(j):
            return out_hbm.at[pl.ds((g0 + j) * _LANE, _LANE)]

        def wait(sem, b, j):
            pltpu.make_async_copy(wbuf_v.at[b], out_slice(j), sem.at[b]).wait()

        fire_w(0, 0)
        fire_w(1, 1)
        fire_p(0, 0)

        def step(j, b, b2):
            bp = (b + 2) % _RING

            @pl.when(j + 2 < g_per_w)
            def _fire_word():
                @pl.when(j >= _RING - 2)
                def _drain_old_write():
                    wait(sem_o, bp, j)
                fire_w(j + 2, bp)

            @pl.when(j + 1 < g_per_w)
            def _fire_pitch():
                fire_p(j + 1, 1 - b2)

            wait(sem_w, b, j)
            wait(sem_p, b2, j)

            def patch(r4, c2):
                for rr in range(4):
                    r = r4 * 4 + rr
                    for kk in range(_D // 16):
                        wbuf_v[b, r, pl.ds(_D + 16 * kk, 16)] = (
                            gp_v[b2, r, pl.ds(16 * kk, 16)])
                return c2

            lax.fori_loop(0, _LANE // 4, patch, 0)
            pltpu.async_copy(wbuf_v.at[b], out_slice(j), sem_o.at[b])

        n_main = (g_per_w // _RING) * _RING

        def body(i, carry):
            for bb in range(_RING):
                step(_RING * i + bb, bb, bb % 2)
            return carry

        lax.fori_loop(0, n_main // _RING, body, 0)
        for j in range(n_main, g_per_w):
            step(j, j % _RING, j % 2)
        for j in range(g_per_w - _RING, g_per_w):
            wait(sem_o, j % _RING, j)

    return k(src_idx, p_idx, tab)


def kernel(src_seq, mask, d_seq, p_seq, src_word_emb, pitch_emb):
    B, L = src_seq.shape
    src_idx = src_seq.reshape(32, -1, _LANE).astype(jnp.int32)
    p_idx = p_seq.reshape(32, -1, _LANE).astype(jnp.int32)
    tab = jnp.pad(jnp.concatenate([src_word_emb, pitch_emb], axis=0),
                  ((0, 0), (0, _D)))
    out = _encoder_gather(src_idx, p_idx, tab, pitch_emb.shape[0])
    return out.reshape(B, L, 2 * _D)
